# baseline (speedup 1.0000x reference)
import functools

import jax
import jax.numpy as jnp
from jax import lax
from jax.experimental import pallas as pl
from jax.experimental.pallas import tpu as pltpu
from jax.experimental.pallas import tpu_sc as plsc

_B, _F, _C = 4096, 26, 1000
_NC, _NS = 2, 16
_NW = _NC * _NS
_BW = _B // _NW
_CW = 128
_NBUF = 4
_CHK = (0, 128, 256, 384, 512, 640, 768, 872)
_CWQ = (128, 128, 128, 128, 128, 128, 104, 128)
_NQ = len(_CHK)


def _onehot_body(xt_hbm, out_hbm, xcol_v, buf_v, s0, s1, s2, s3):
    sems = (s0, s1, s2, s3)
    wid = lax.axis_index("c") * _NS + lax.axis_index("s")
    b0 = wid * _BW

    zeros = jnp.zeros((16,), jnp.float32)
    ones = jnp.ones((16,), jnp.float32)
    lanes = lax.iota(jnp.int32, 16)

    pltpu.sync_copy(xt_hbm.at[:, pl.ds(b0, _BW)], xcol_v)

    def _zero_slot(s):
        def _zrow(t, carry):
            buf_v[s, t // 8, pl.ds((t % 8) * 16, 16)] = zeros
            return carry
        lax.fori_loop(0, _CW * (_BW // 16), _zrow, 0)

    def _scatter(s, q, f, val):
        c0 = _CHK[q]
        sv = jnp.full((16,), s, jnp.int32)

        def _grp(u, carry):
            cj = xcol_v[f, pl.ds(u * 16, 16)]
            m = (cj >= c0) & (cj < c0 + _CWQ[q])
            plsc.store_scatter(buf_v, [sv, cj - c0, u * 16 + lanes],
                               val, mask=m)
            return carry

        lax.fori_loop(0, _BW // 16, _grp, 0)

    def _clear_set(s, oldq, of, q, f):
        oc0, nc0 = _CHK[oldq], _CHK[q]
        sv = jnp.full((16,), s, jnp.int32)

        def _grp(u, carry):
            oj = xcol_v[of, pl.ds(u * 16, 16)]
            om = (oj >= oc0) & (oj < oc0 + _CWQ[oldq])
            plsc.store_scatter(buf_v, [sv, oj - oc0, u * 16 + lanes],
                               zeros, mask=om)
            nj = xcol_v[f, pl.ds(u * 16, 16)]
            nm = (nj >= nc0) & (nj < nc0 + _CWQ[q])
            plsc.store_scatter(buf_v, [sv, nj - nc0, u * 16 + lanes],
                               ones, mask=nm)
            return carry

        lax.fori_loop(0, _BW // 16, _grp, 0)

    def _copy(s, f, q):
        return pltpu.make_async_copy(
            buf_v.at[s, pl.ds(0, _CWQ[q])],
            out_hbm.at[f, pl.ds(_CHK[q], _CWQ[q]), pl.ds(b0, _BW)],
            sems[s])

    for q in range(_NBUF):
        _zero_slot(q)
        _scatter(q, q, 0, ones)
        _copy(q, 0, q).start()
    for q in range(_NBUF, _NQ):
        s = q % _NBUF
        _copy(s, 0, q - _NBUF).wait()
        _clear_set(s, q - _NBUF, 0, q, 0)
        _copy(s, 0, q).start()

    def _body(f, carry):
        for q in range(_NQ):
            s = q % _NBUF
            oldq = (q + _NBUF) % _NQ
            of = f - 1 if q < _NBUF else f
            _copy(s, of, oldq).wait()
            _clear_set(s, oldq, of, q, f)
            _copy(s, f, q).start()
        return carry

    lax.fori_loop(1, _F, _body, 0)

    for q in range(_NBUF, _NQ):
        _copy(q % _NBUF, _F - 1, q).wait()


_onehot_sc = functools.partial(
    pl.kernel,
    out_type=jax.ShapeDtypeStruct((_F, _C, _B), jnp.float32),
    mesh=plsc.VectorSubcoreMesh(core_axis_name="c", subcore_axis_name="s"),
    compiler_params=pltpu.CompilerParams(needs_layout_passes=False),
    scratch_types=[
        pltpu.VMEM((_F, _BW), jnp.int32),
        pltpu.VMEM((_NBUF, _CW, _BW), jnp.float32),
        pltpu.SemaphoreType.DMA,
        pltpu.SemaphoreType.DMA,
        pltpu.SemaphoreType.DMA,
        pltpu.SemaphoreType.DMA,
    ],
)(_onehot_body)


def kernel(x):
    xt = x.astype(jnp.int32).T
    return _onehot_sc(xt).transpose(2, 0, 1)


# speedup vs baseline: 1.0855x; 1.0855x over previous
import functools

import jax
import jax.numpy as jnp
from jax import lax
from jax.experimental import pallas as pl
from jax.experimental.pallas import tpu as pltpu
from jax.experimental.pallas import tpu_sc as plsc

_B, _F, _C = 4096, 26, 1000
_NC, _NS = 2, 16
_NW = _NC * _NS
_BW = _B // _NW
_CW = 128
_NBUF = 4
_CHK = (0, 128, 256, 384, 512, 640, 768, 872)
_CWQ = (128, 128, 128, 128, 128, 128, 104, 128)
_NQ = len(_CHK)


def _onehot_body(xt_hbm, out_hbm, xcol_v, buf_v, s0, s1, s2, s3):
    sems = (s0, s1, s2, s3)
    wid = lax.axis_index("c") * _NS + lax.axis_index("s")
    b0 = wid * _BW

    zeros = jnp.zeros((16,), jnp.float32)
    ones = jnp.ones((16,), jnp.float32)
    lanes = lax.iota(jnp.int32, 16)

    pltpu.sync_copy(xt_hbm.at[:, pl.ds(b0, _BW)], xcol_v)

    def _zero_slot(s):
        def _zrow(r, carry):
            for u in range(_BW // 16):
                buf_v[s, r, pl.ds(u * 16, 16)] = zeros
            return carry
        lax.fori_loop(0, _CW, _zrow, 0)

    def _scatter(s, q, f, val):
        c0 = _CHK[q]
        sv = jnp.full((16,), s, jnp.int32)

        def _grp(u, carry):
            cj = xcol_v[f, pl.ds(u * 16, 16)]
            m = (cj >= c0) & (cj < c0 + _CWQ[q])
            plsc.store_scatter(buf_v, [sv, cj - c0, u * 16 + lanes],
                               val, mask=m)
            return carry

        lax.fori_loop(0, _BW // 16, _grp, 0)

    def _copy(s, f, q):
        return pltpu.make_async_copy(
            buf_v.at[s, pl.ds(0, _CWQ[q])],
            out_hbm.at[f, pl.ds(_CHK[q], _CWQ[q]), pl.ds(b0, _BW)],
            sems[s])

    for q in range(_NBUF):
        _zero_slot(q)
        _scatter(q, q, 0, ones)
        _copy(q, 0, q).start()
    for q in range(_NBUF, _NQ):
        s = q % _NBUF
        _copy(s, 0, q - _NBUF).wait()
        _scatter(s, q - _NBUF, 0, zeros)
        _scatter(s, q, 0, ones)
        _copy(s, 0, q).start()

    def _body(f, carry):
        for q in range(_NQ):
            s = q % _NBUF
            oldq = (q + _NBUF) % _NQ
            of = f - 1 if q < _NBUF else f
            _copy(s, of, oldq).wait()
            _scatter(s, oldq, of, zeros)
            _scatter(s, q, f, ones)
            _copy(s, f, q).start()
        return carry

    lax.fori_loop(1, _F, _body, 0)

    for q in range(_NBUF, _NQ):
        _copy(q % _NBUF, _F - 1, q).wait()


_onehot_sc = functools.partial(
    pl.kernel,
    out_type=jax.ShapeDtypeStruct((_F, _C, _B), jnp.float32),
    mesh=plsc.VectorSubcoreMesh(core_axis_name="c", subcore_axis_name="s"),
    compiler_params=pltpu.CompilerParams(needs_layout_passes=False),
    scratch_types=[
        pltpu.VMEM((_F, _BW), jnp.int32),
        pltpu.VMEM((_NBUF, _CW, _BW), jnp.float32),
        pltpu.SemaphoreType.DMA,
        pltpu.SemaphoreType.DMA,
        pltpu.SemaphoreType.DMA,
        pltpu.SemaphoreType.DMA,
    ],
)(_onehot_body)


def kernel(x):
    xt = x.astype(jnp.int32).T
    return _onehot_sc(xt).transpose(2, 0, 1)
